# baseline (speedup 1.0000x reference)
import functools

import jax
import jax.numpy as jnp
from jax import lax
from jax.experimental import pallas as pl
from jax.experimental.pallas import tpu as pltpu
from jax.experimental.pallas import tpu_sc as plsc

_DP = 32
_GB = 4

_INFO = plsc.get_sparse_core_info()
_NC = _INFO.num_cores
_NS = _INFO.num_subcores
_NW = _NC * _NS


@functools.cache
def _build(B, S):
  rows_per_w = B // _NW
  ng = rows_per_w // _GB
  chunks = []
  s0 = 0
  while s0 < S:
    c = min(128, S - s0)
    chunks.append((s0, c))
    s0 += c
  mesh = plsc.VectorSubcoreMesh(core_axis_name="c", subcore_axis_name="s")

  @functools.partial(
      pl.kernel, mesh=mesh,
      out_type=jax.ShapeDtypeStruct((B, S, _DP), jnp.float32),
      compiler_params=pltpu.CompilerParams(use_tc_tiling_on_sc=False),
      scratch_types=[
          pltpu.VMEM((_GB, S), jnp.int32),
          pltpu.VMEM((_GB, S, _DP), jnp.float32),
          pltpu.SemaphoreType.DMA,
      ],
  )
  def gather_kernel(idx_hbm, table_hbm, out_hbm, idx_v, rows_v, sem):
    wid = lax.axis_index("s") * _NC + lax.axis_index("c")
    base = wid * rows_per_w

    def step(g, carry):
      b0 = base + g * _GB
      pltpu.sync_copy(idx_hbm.at[pl.ds(b0, _GB)], idx_v)
      copies = []
      for bi in range(_GB):
        for (s0, c) in chunks:
          copies.append(pltpu.async_copy(
              table_hbm.at[idx_v.at[bi, pl.ds(s0, c)]],
              rows_v.at[bi, pl.ds(s0, c)], sem))
      for cp in copies:
        cp.wait()
      pltpu.sync_copy(rows_v, out_hbm.at[pl.ds(b0, _GB)])
      return carry

    lax.fori_loop(0, ng, step, 0)

  return gather_kernel


def kernel(indices, table):
  B, S = indices.shape
  V, D = table.shape
  idx = indices.astype(jnp.int32)
  tab = lax.optimization_barrier(jnp.pad(table, ((0, 0), (0, _DP - D))))
  out = _build(B, S)(idx, tab)
  out2d = out.reshape(B, S * _DP)
  stripped = jnp.concatenate(
      [out2d[:, s * _DP:s * _DP + D] for s in range(S)], axis=1)
  stripped = lax.optimization_barrier(stripped)
  return stripped.reshape(B, S, D)


# speedup vs baseline: 1.6569x; 1.6569x over previous
import functools

import jax
import jax.numpy as jnp
from jax import lax
from jax.experimental import pallas as pl
from jax.experimental.layout import Format, Layout
from jax.experimental.pallas import tpu as pltpu
from jax.experimental.pallas import tpu_sc as plsc

_DP = 32
_GB = 4

_INFO = plsc.get_sparse_core_info()
_NC = _INFO.num_cores
_NS = _INFO.num_subcores
_NW = _NC * _NS


@functools.cache
def _build(B, S):
  rows_per_w = B // _NW
  ng = rows_per_w // _GB
  chunks = []
  s0 = 0
  while s0 < S:
    c = min(128, S - s0)
    chunks.append((s0, c))
    s0 += c
  mesh = plsc.VectorSubcoreMesh(core_axis_name="c", subcore_axis_name="s")

  @functools.partial(
      pl.kernel, mesh=mesh,
      out_type=jax.ShapeDtypeStruct((B, S, _DP), jnp.float32),
      compiler_params=pltpu.CompilerParams(use_tc_tiling_on_sc=False),
      scratch_types=[
          pltpu.VMEM((_GB, S), jnp.int32),
          pltpu.VMEM((_GB, S, _DP), jnp.float32),
          pltpu.SemaphoreType.DMA,
      ],
  )
  def gather_kernel(idx_hbm, table_hbm, out_hbm, idx_v, rows_v, sem):
    wid = lax.axis_index("s") * _NC + lax.axis_index("c")
    base = wid * rows_per_w

    def step(g, carry):
      b0 = base + g * _GB
      pltpu.sync_copy(idx_hbm.at[pl.ds(b0, _GB)], idx_v)
      copies = []
      for bi in range(_GB):
        for (s0, c) in chunks:
          copies.append(pltpu.async_copy(
              table_hbm.at[idx_v.at[bi, pl.ds(s0, c)]],
              rows_v.at[bi, pl.ds(s0, c)], sem))
      for cp in copies:
        cp.wait()
      pltpu.sync_copy(rows_v, out_hbm.at[pl.ds(b0, _GB)])
      return carry

    lax.fori_loop(0, ng, step, 0)

  return gather_kernel


def kernel(indices, table):
  B, S = indices.shape
  V, D = table.shape
  idx = indices.astype(jnp.int32)
  dev = jax.sharding.SingleDeviceSharding(jax.devices()[0])
  compact2d = Format(Layout(major_to_minor=(1, 0), tiling=((8, 128),)), dev)
  pad_pinned = jax.jit(
      lambda t: jnp.pad(t, ((0, 0), (0, _DP - D))),
      in_shardings=compact2d, out_shardings=compact2d)
  tab = pad_pinned(table)
  out = _build(B, S)(idx, tab)
  compact3d = Format(Layout(major_to_minor=(2, 1, 0), tiling=((8, 128),)), dev)
  strip_pinned = jax.jit(lambda o: o[:, :, :D], out_shardings=compact3d)
  return strip_pinned(out)


# speedup vs baseline: 1.8966x; 1.1447x over previous
import functools

import jax
import jax.numpy as jnp
from jax import lax
from jax.experimental import pallas as pl
from jax.experimental.pallas import tpu as pltpu
from jax.experimental.pallas import tpu_sc as plsc

_DP = 32
_GB = 4

_INFO = plsc.get_sparse_core_info()
_NC = _INFO.num_cores
_NS = _INFO.num_subcores
_NW = _NC * _NS


@functools.cache
def _build(B, S):
  rows_per_w = B // _NW
  ng = rows_per_w // _GB
  chunks = []
  s0 = 0
  while s0 < S:
    c = min(128, S - s0)
    chunks.append((s0, c))
    s0 += c
  mesh = plsc.VectorSubcoreMesh(core_axis_name="c", subcore_axis_name="s")

  @functools.partial(
      pl.kernel, mesh=mesh,
      out_type=jax.ShapeDtypeStruct((B, S, _DP), jnp.float32),
      compiler_params=pltpu.CompilerParams(use_tc_tiling_on_sc=False),
      scratch_types=[
          pltpu.VMEM((_GB, S), jnp.int32),
          pltpu.VMEM((_GB, S, _DP), jnp.float32),
          pltpu.SemaphoreType.DMA,
      ],
  )
  def gather_kernel(idx_hbm, table_hbm, out_hbm, idx_v, rows_v, sem):
    wid = lax.axis_index("s") * _NC + lax.axis_index("c")
    base = wid * rows_per_w

    def step(g, carry):
      b0 = base + g * _GB
      pltpu.sync_copy(idx_hbm.at[pl.ds(b0, _GB)], idx_v)
      copies = []
      for bi in range(_GB):
        for (s0, c) in chunks:
          copies.append(pltpu.async_copy(
              table_hbm.at[idx_v.at[bi, pl.ds(s0, c)]],
              rows_v.at[bi, pl.ds(s0, c)], sem))
      for cp in copies:
        cp.wait()
      pltpu.sync_copy(rows_v, out_hbm.at[pl.ds(b0, _GB)])
      return carry

    lax.fori_loop(0, ng, step, 0)

  return gather_kernel


def kernel(indices, table):
  B, S = indices.shape
  V, D = table.shape
  idx = indices.astype(jnp.int32)
  tab = jnp.concatenate([table, table[:, :_DP - D]], axis=1)
  out = _build(B, S)(idx, tab)
  return out[:, :, :D]


# speedup vs baseline: 1.9726x; 1.0401x over previous
import functools

import jax
import jax.numpy as jnp
from jax import lax
from jax.experimental import pallas as pl
from jax.experimental.pallas import tpu as pltpu
from jax.experimental.pallas import tpu_sc as plsc

_DP = 32
_GB = 4

_INFO = plsc.get_sparse_core_info()
_NC = _INFO.num_cores
_NS = _INFO.num_subcores
_NW = _NC * _NS


@functools.cache
def _build(B, S):
  rows_per_w = B // _NW
  ng = rows_per_w // _GB
  chunks = []
  s0 = 0
  while s0 < S:
    c = min(128, S - s0)
    chunks.append((s0, c))
    s0 += c
  mesh = plsc.VectorSubcoreMesh(core_axis_name="c", subcore_axis_name="s")

  @functools.partial(
      pl.kernel, mesh=mesh,
      out_type=jax.ShapeDtypeStruct((B, S, _DP), jnp.float32),
      compiler_params=pltpu.CompilerParams(use_tc_tiling_on_sc=False),
      scratch_types=[
          pltpu.VMEM((2, _GB, S), jnp.int32),
          pltpu.VMEM((2, _GB, S, _DP), jnp.float32),
          pltpu.SemaphoreType.DMA,
          pltpu.SemaphoreType.DMA,
      ],
  )
  def gather_kernel(idx_hbm, table_hbm, out_hbm, idx_v, rows_v, semg, semo):
    wid = lax.axis_index("s") * _NC + lax.axis_index("c")
    base = wid * rows_per_w

    def fire(g, p):
      b0 = base + g * _GB
      pltpu.sync_copy(idx_hbm.at[pl.ds(b0, _GB)], idx_v.at[p])
      copies = []
      for bi in range(_GB):
        for (s0, c) in chunks:
          copies.append(pltpu.async_copy(
              table_hbm.at[idx_v.at[p, bi, pl.ds(s0, c)]],
              rows_v.at[p, bi, pl.ds(s0, c)], semg))
      return copies

    def drain_and_store(g, p, copies):
      for cp in copies:
        cp.wait()
      b0 = base + g * _GB
      return pltpu.async_copy(rows_v.at[p], out_hbm.at[pl.ds(b0, _GB)], semo)

    def step(k, carry):
      ga, gb = 2 * k, 2 * k + 1
      ca = fire(ga, 0)
      cb = fire(gb, 1)
      oa = drain_and_store(ga, 0, ca)
      ob = drain_and_store(gb, 1, cb)
      oa.wait()
      ob.wait()
      return carry

    lax.fori_loop(0, ng // 2, step, 0)

  return gather_kernel


def kernel(indices, table):
  B, S = indices.shape
  V, D = table.shape
  idx = indices.astype(jnp.int32)
  tab = jnp.concatenate([table, table[:, :_DP - D]], axis=1)
  out = _build(B, S)(idx, tab)
  return out[:, :, :D]


# speedup vs baseline: 1.9727x; 1.0001x over previous
import functools

import jax
import jax.numpy as jnp
from jax import lax
from jax.experimental import pallas as pl
from jax.experimental.pallas import tpu as pltpu
from jax.experimental.pallas import tpu_sc as plsc

_DP = 32
_GB = 4

_INFO = plsc.get_sparse_core_info()
_NC = _INFO.num_cores
_NS = _INFO.num_subcores
_NW = _NC * _NS


@functools.cache
def _build(B, S):
  rows_per_w = B // _NW
  ng = rows_per_w // _GB
  chunks = []
  s0 = 0
  while s0 < S:
    c = min(128, S - s0)
    chunks.append((s0, c))
    s0 += c
  mesh = plsc.VectorSubcoreMesh(core_axis_name="c", subcore_axis_name="s")

  @functools.partial(
      pl.kernel, mesh=mesh,
      out_type=jax.ShapeDtypeStruct((B, S, _DP), jnp.float32),
      compiler_params=pltpu.CompilerParams(use_tc_tiling_on_sc=False),
      scratch_types=[
          pltpu.VMEM((2, _GB, S), jnp.int32),
          pltpu.VMEM((2, _GB, S, _DP), jnp.float32),
          pltpu.SemaphoreType.DMA,
          pltpu.SemaphoreType.DMA,
      ],
  )
  def gather_kernel(idx_hbm, table_hbm, out_hbm, idx_v, rows_v, semg, semo):
    wid = lax.axis_index("s") * _NC + lax.axis_index("c")
    base = wid * rows_per_w

    def fire(g, p):
      b0 = base + g * _GB
      pltpu.sync_copy(idx_hbm.at[pl.ds(b0, _GB)], idx_v.at[p])
      copies = []
      for bi in range(_GB):
        for (s0, c) in chunks:
          copies.append(pltpu.async_copy(
              table_hbm.at[idx_v.at[p, bi, pl.ds(s0, c)]],
              rows_v.at[p, bi, pl.ds(s0, c)], semg))
      return copies

    def drain_and_store(g, p, copies):
      for cp in copies:
        cp.wait()
      b0 = base + g * _GB
      return pltpu.async_copy(rows_v.at[p], out_hbm.at[pl.ds(b0, _GB)], semo)

    def drain_outputs():
      for p in range(2):
        pltpu.make_async_copy(
            rows_v.at[p], out_hbm.at[pl.ds(base, _GB)], semo).wait()

    def step(k, carry):
      ga, gb = 2 * k, 2 * k + 1

      @pl.when(k > 0)
      def _():
        drain_outputs()

      ca = fire(ga, 0)
      cb = fire(gb, 1)
      drain_and_store(ga, 0, ca)
      drain_and_store(gb, 1, cb)
      return carry

    lax.fori_loop(0, ng // 2, step, 0)
    drain_outputs()

  return gather_kernel


def kernel(indices, table):
  B, S = indices.shape
  V, D = table.shape
  idx = indices.astype(jnp.int32)
  tab = jnp.concatenate([table, table[:, :_DP - D]], axis=1)
  out = _build(B, S)(idx, tab)
  return out[:, :, :D]
